# baseline (speedup 1.0000x reference)
import functools

import jax
import jax.numpy as jnp
import numpy as np
from jax import lax
from jax.experimental import pallas as pl
from jax.experimental.pallas import tpu as pltpu
from jax.experimental.pallas import tpu_sc as plsc

_ALPHA = 0.2
_NPAD = 1024
_ACCR = _NPAD + 512
_CH = 128


def _sc_body(nc, ns, e_w, nch,
             xT, idxg, idxs, w0f, w1f, ddf, zx,
             outx0, outx1, outd,
             rg_v, r1_v, msgd_v, idxg_v, idxs_v, w0_v, w1_v, dd_v,
             stage_v, pack_v,
             acc0_s, acc1_s, accd_s, gsem, ssem):
    cid = lax.axis_index("c")
    sid = lax.axis_index("s")
    wid = sid * nc + cid

    pltpu.sync_copy(idxg.at[wid], idxg_v)
    pltpu.sync_copy(idxs.at[wid], idxs_v)
    pltpu.sync_copy(w0f.at[wid], w0_v)
    pltpu.sync_copy(w1f.at[wid], w1_v)
    pltpu.sync_copy(ddf.at[wid], dd_v)

    rpw = _NPAD // ns
    pltpu.sync_copy(zx, acc0_s.at[pl.ds(sid * rpw, rpw)])
    pltpu.sync_copy(zx, acc1_s.at[pl.ds(sid * rpw, rpw)])
    pltpu.sync_copy(zx, accd_s.at[pl.ds(sid * rpw, rpw)])

    gcps = [
        pltpu.async_copy(xT.at[idxg_v.at[c]],
                         rg_v.at[pl.ds(c * _CH, _CH)], gsem)
        for c in range(nch)
    ]
    plsc.subcore_barrier()

    lane = lax.iota(jnp.int32, 16)
    m01 = lane == 0
    zv = jnp.zeros((16,), jnp.float32)
    s0, s1 = pl.ds(0, 16), pl.ds(16, 16)
    scps = []
    for c in range(nch):
        gcps[c].wait()

        @plsc.parallel_loop(0, _CH, step=1, unroll=2)
        def mul_body(k, _c=c):
            e = _c * _CH + k
            ei = jnp.full((16,), e, jnp.int32)
            w0b = plsc.load_gather(w0_v, [ei])
            w1b = plsc.load_gather(w1_v, [ei])
            ddb = plsc.load_gather(dd_v, [ei])
            a = rg_v[e, s0]
            b_ = rg_v[e, s1]
            r1_v[e, s0] = w1b * a
            r1_v[e, s1] = w1b * b_
            rg_v[e, s0] = w0b * a
            rg_v[e, s1] = w0b * b_
            msgd_v[e, s0] = jnp.where(m01, w0b, w1b) * ddb
            msgd_v[e, s1] = zv

        scps.append(pltpu.async_copy(
            rg_v.at[pl.ds(c * _CH, _CH)], acc0_s.at[idxs_v.at[c]],
            ssem, add=True))
        scps.append(pltpu.async_copy(
            r1_v.at[pl.ds(c * _CH, _CH)], acc1_s.at[idxs_v.at[c]],
            ssem, add=True))
        scps.append(pltpu.async_copy(
            msgd_v.at[pl.ds(c * _CH, _CH)], accd_s.at[idxs_v.at[c]],
            ssem, add=True))
    for cp in scps:
        cp.wait()

    plsc.subcore_barrier()

    rows = rpw * 32 // 128
    src = pl.ds(sid * rpw, rpw)
    dst = pl.ds(cid * (_NPAD * 32 // 128) + sid * rows, rows)
    for acc_s, out_ref in ((acc0_s, outx0), (acc1_s, outx1),
                           (accd_s, outd)):
        pltpu.sync_copy(acc_s.at[src], stage_v)

        @plsc.parallel_loop(0, rpw * 2, step=1, unroll=4)
        def pack_body(v):
            val = stage_v[v // 2, pl.ds((v % 2) * 16, 16)]
            pack_v[v // 8, pl.ds((v % 8) * 16, 16)] = val

        pltpu.sync_copy(pack_v, out_ref.at[dst])


def _fc_body(nc, x0_ref, x1_ref, d_ref, A0_ref, A1_ref, Ad_ref, bt_ref,
             out_ref):
    hp = _NPAD * 32 // 128
    x0p = x0_ref[:hp, :] + x0_ref[hp:, :]
    x1p = x1_ref[:hp, :] + x1_ref[hp:, :]
    dp = d_ref[:hp, :] + d_ref[hp:, :]
    A0 = A0_ref[...]
    A1 = A1_ref[...]
    Ad = Ad_ref[...]
    for i in range(4):
        s = slice(32 * i, 32 * i + 32)
        y = (jnp.dot(x0p[:, s], A0, preferred_element_type=jnp.float32)
             + jnp.dot(x1p[:, s], A1, preferred_element_type=jnp.float32)
             + jnp.dot(dp[:, s], Ad, preferred_element_type=jnp.float32)
             + bt_ref[0:1, :])
        for t in range(4):
            o = jnp.concatenate(
                [y[:, 8 * t:8 * t + 8], y[:, 32 + 8 * t:32 + 8 * t + 8]],
                axis=1)
            out_ref[t, :, pl.ds(16 * i, 16)] = 1.0 / (1.0 + jnp.exp(-o))


def kernel(x, T, d_ew, d_edges, d_dist, W, b):
    del T
    _, T_, N, Cx = x.shape
    E = d_edges.shape[0]
    F = T_ * Cx

    info = plsc.get_sparse_core_info()
    nc, ns = info.num_cores, info.num_subcores
    nw = nc * ns
    e_w = -(-E // (nw * _CH)) * _CH
    nch = e_w // _CH
    e_pad = nw * e_w

    xT = x[0].transpose(1, 0, 2).reshape(N, F)
    xTp = jnp.zeros((_NPAD, F), jnp.float32).at[:N].set(xT)

    ar = jnp.arange(e_pad, dtype=jnp.int32)
    ni = (ar % _NPAD).at[:E].set(d_edges[:, 0])
    nj = (_NPAD + (ar % (_ACCR - _NPAD))).at[:E].set(d_edges[:, 1])
    w0 = jnp.zeros((e_pad,), jnp.float32).at[:E].set(d_ew[:, 0])
    w1 = jnp.zeros((e_pad,), jnp.float32).at[:E].set(d_ew[:, 1])
    dd = jnp.zeros((e_pad,), jnp.float32).at[:E].set(d_dist)

    idxg = ni.reshape(nw, nch, _CH)
    idxs = nj.reshape(nw, nch, _CH)
    w0f = w0.reshape(nw, e_w)
    w1f = w1.reshape(nw, e_w)
    ddf = dd.reshape(nw, e_w)
    rpw = _NPAD // ns
    zx = jnp.zeros((rpw, F), jnp.float32)

    orows = nc * _NPAD * F // 128
    mesh = plsc.VectorSubcoreMesh(core_axis_name="c", subcore_axis_name="s")
    outx0, outx1, outd = pl.kernel(
        functools.partial(_sc_body, nc, ns, e_w, nch),
        out_type=[
            jax.ShapeDtypeStruct((orows, 128), jnp.float32),
            jax.ShapeDtypeStruct((orows, 128), jnp.float32),
            jax.ShapeDtypeStruct((orows, 128), jnp.float32),
        ],
        mesh=mesh,
        compiler_params=pltpu.CompilerParams(
            use_tc_tiling_on_sc=False, needs_layout_passes=False),
        scratch_types=[
            pltpu.VMEM((e_w, F), jnp.float32),
            pltpu.VMEM((e_w, F), jnp.float32),
            pltpu.VMEM((e_w, F), jnp.float32),
            pltpu.VMEM((nch, _CH), jnp.int32),
            pltpu.VMEM((nch, _CH), jnp.int32),
            pltpu.VMEM((e_w,), jnp.float32),
            pltpu.VMEM((e_w,), jnp.float32),
            pltpu.VMEM((e_w,), jnp.float32),
            pltpu.VMEM((_NPAD // 16, F), jnp.float32),
            pltpu.VMEM((_NPAD * F // 16 // 128, 128), jnp.float32),
            pltpu.VMEM_SHARED((_ACCR, F), jnp.float32),
            pltpu.VMEM_SHARED((_ACCR, F), jnp.float32),
            pltpu.VMEM_SHARED((_ACCR, F), jnp.float32),
            pltpu.SemaphoreType.DMA,
            pltpu.SemaphoreType.DMA,
        ],
    )(xTp, idxg, idxs, w0f, w1f, ddf, zx)

    M = np.zeros((4, 4), np.float32)
    M[0, 0] = 1.0
    for t in range(1, 4):
        M[t, t] = 1.0 - _ALPHA
        M[t - 1, t] = _ALPHA
    S32 = jnp.asarray(np.kron(M, np.eye(8, dtype=np.float32)))
    KWS = jnp.kron(jnp.eye(4, dtype=jnp.float32), W[:8, :]) @ S32
    Z32 = jnp.zeros((32, 32), jnp.float32)
    A0 = jnp.concatenate([KWS, Z32], axis=1)
    A1 = jnp.concatenate([Z32, KWS], axis=1)
    w8s = jnp.tile(W[8, :], 4) @ S32
    Ad = jnp.zeros((32, 64), jnp.float32).at[0, :32].set(w8s)
    Ad = Ad.at[1, 32:].set(w8s)
    bt = jnp.broadcast_to(jnp.tile(b, 8)[None, :], (8, 64))

    out = pl.pallas_call(
        functools.partial(_fc_body, nc),
        in_specs=[
            pl.BlockSpec((orows, 128), lambda: (0, 0)),
            pl.BlockSpec((orows, 128), lambda: (0, 0)),
            pl.BlockSpec((orows, 128), lambda: (0, 0)),
            pl.BlockSpec((32, 64), lambda: (0, 0)),
            pl.BlockSpec((32, 64), lambda: (0, 0)),
            pl.BlockSpec((32, 64), lambda: (0, 0)),
            pl.BlockSpec((8, 64), lambda: (0, 0)),
        ],
        out_specs=pl.BlockSpec((T_, _NPAD // 4, 64), lambda: (0, 0, 0)),
        out_shape=jax.ShapeDtypeStruct((T_, _NPAD // 4, 64), jnp.float32),
    )(outx0, outx1, outd, A0, A1, Ad, bt)

    res = out.reshape(T_, _NPAD, 2, 8)[:, :N]
    return res[None]


# speedup vs baseline: 1.3195x; 1.3195x over previous
import functools

import jax
import jax.numpy as jnp
import numpy as np
from jax import lax
from jax.experimental import pallas as pl
from jax.experimental.pallas import tpu as pltpu
from jax.experimental.pallas import tpu_sc as plsc

_ALPHA = 0.2
_NPAD = 1024
_ACCR = _NPAD + 512
_CH = 128


def _sc_body(nc, ns, e_w, nch,
             xT, idxg, idxs, w0f, w1f, ddf, zx,
             outx0, outx1, outd,
             rg_v, r1_v, msgd_v, idxg_v, idxs_v, w0_v, w1_v, dd_v,
             stage_v, pack_v,
             acc0_s, acc1_s, accd_s, gsem, ssem):
    cid = lax.axis_index("c")
    sid = lax.axis_index("s")
    wid = sid * nc + cid

    pltpu.sync_copy(idxg.at[wid], idxg_v)
    pltpu.sync_copy(idxs.at[wid], idxs_v)
    pltpu.sync_copy(w0f.at[wid], w0_v)
    pltpu.sync_copy(w1f.at[wid], w1_v)
    pltpu.sync_copy(ddf.at[wid], dd_v)

    rpw = _NPAD // ns
    pltpu.sync_copy(zx, acc0_s.at[pl.ds(sid * rpw, rpw)])
    pltpu.sync_copy(zx, acc1_s.at[pl.ds(sid * rpw, rpw)])
    pltpu.sync_copy(zx, accd_s.at[pl.ds(sid * rpw, rpw)])

    gcps = [
        pltpu.async_copy(xT.at[idxg_v.at[c]],
                         rg_v.at[pl.ds(c * _CH, _CH)], gsem)
        for c in range(nch)
    ]
    plsc.subcore_barrier()

    lane = lax.iota(jnp.int32, 16)
    m01 = lane == 0
    zv = jnp.zeros((16,), jnp.float32)
    s0, s1 = pl.ds(0, 16), pl.ds(16, 16)
    scps = []
    for c in range(nch):
        gcps[c].wait()

        @plsc.parallel_loop(0, _CH, step=1, unroll=2)
        def mul_body(k, _c=c):
            e = _c * _CH + k
            ei = jnp.full((16,), e, jnp.int32)
            w0b = plsc.load_gather(w0_v, [ei])
            w1b = plsc.load_gather(w1_v, [ei])
            ddb = plsc.load_gather(dd_v, [ei])
            a = rg_v[e, s0]
            b_ = rg_v[e, s1]
            r1_v[e, s0] = w1b * a
            r1_v[e, s1] = w1b * b_
            rg_v[e, s0] = w0b * a
            rg_v[e, s1] = w0b * b_
            msgd_v[e, s0] = jnp.where(m01, w0b, w1b) * ddb
            msgd_v[e, s1] = zv

        scps.append(pltpu.async_copy(
            rg_v.at[pl.ds(c * _CH, _CH)], acc0_s.at[idxs_v.at[c]],
            ssem, add=True))
        scps.append(pltpu.async_copy(
            r1_v.at[pl.ds(c * _CH, _CH)], acc1_s.at[idxs_v.at[c]],
            ssem, add=True))
        scps.append(pltpu.async_copy(
            msgd_v.at[pl.ds(c * _CH, _CH)], accd_s.at[idxs_v.at[c]],
            ssem, add=True))
    for cp in scps:
        cp.wait()

    plsc.subcore_barrier()

    rows = rpw * 32 // 128
    src = pl.ds(sid * rpw, rpw)
    dst = pl.ds(cid * (_NPAD * 32 // 128) + sid * rows, rows)
    for acc_s, out_ref in ((acc0_s, outx0), (acc1_s, outx1),
                           (accd_s, outd)):
        pltpu.sync_copy(acc_s.at[src], stage_v)

        @plsc.parallel_loop(0, rpw * 2, step=1, unroll=4)
        def pack_body(v):
            val = stage_v[v // 2, pl.ds((v % 2) * 16, 16)]
            pack_v[v // 8, pl.ds((v % 8) * 16, 16)] = val

        pltpu.sync_copy(pack_v, out_ref.at[dst])


def _fc_body(nc, x0_ref, x1_ref, d_ref, A0_ref, A1_ref, Ad_ref, bt_ref,
             out_ref):
    hp = _NPAD * 32 // 128
    x0p = x0_ref[:hp, :] + x0_ref[hp:, :]
    x1p = x1_ref[:hp, :] + x1_ref[hp:, :]
    dp = d_ref[:hp, :] + d_ref[hp:, :]
    A0 = A0_ref[...]
    A1 = A1_ref[...]
    Ad = Ad_ref[...]
    for i in range(4):
        s = slice(32 * i, 32 * i + 32)
        y = (jnp.dot(x0p[:, s], A0, preferred_element_type=jnp.float32)
             + jnp.dot(x1p[:, s], A1, preferred_element_type=jnp.float32)
             + jnp.dot(dp[:, s], Ad, preferred_element_type=jnp.float32)
             + bt_ref[0:1, :])
        for t in range(4):
            o = jnp.concatenate(
                [y[:, 8 * t:8 * t + 8], y[:, 32 + 8 * t:32 + 8 * t + 8]],
                axis=1)
            out_ref[t, pl.Slice(i, _NPAD // 4, 4), :] = (
                1.0 / (1.0 + jnp.exp(-o)))


def kernel(x, T, d_ew, d_edges, d_dist, W, b):
    del T
    _, T_, N, Cx = x.shape
    E = d_edges.shape[0]
    F = T_ * Cx

    info = plsc.get_sparse_core_info()
    nc, ns = info.num_cores, info.num_subcores
    nw = nc * ns
    e_w = -(-E // (nw * _CH)) * _CH
    nch = e_w // _CH
    e_pad = nw * e_w

    xT = x[0].transpose(1, 0, 2).reshape(N, F)
    xTp = jnp.zeros((_NPAD, F), jnp.float32).at[:N].set(xT)

    ar = jnp.arange(e_pad, dtype=jnp.int32)
    ni = (ar % _NPAD).at[:E].set(d_edges[:, 0])
    nj = (_NPAD + (ar % (_ACCR - _NPAD))).at[:E].set(d_edges[:, 1])
    w0 = jnp.zeros((e_pad,), jnp.float32).at[:E].set(d_ew[:, 0])
    w1 = jnp.zeros((e_pad,), jnp.float32).at[:E].set(d_ew[:, 1])
    dd = jnp.zeros((e_pad,), jnp.float32).at[:E].set(d_dist)

    idxg = ni.reshape(nw, nch, _CH)
    idxs = nj.reshape(nw, nch, _CH)
    w0f = w0.reshape(nw, e_w)
    w1f = w1.reshape(nw, e_w)
    ddf = dd.reshape(nw, e_w)
    rpw = _NPAD // ns
    zx = jnp.zeros((rpw, F), jnp.float32)

    orows = nc * _NPAD * F // 128
    mesh = plsc.VectorSubcoreMesh(core_axis_name="c", subcore_axis_name="s")
    outx0, outx1, outd = pl.kernel(
        functools.partial(_sc_body, nc, ns, e_w, nch),
        out_type=[
            jax.ShapeDtypeStruct((orows, 128), jnp.float32),
            jax.ShapeDtypeStruct((orows, 128), jnp.float32),
            jax.ShapeDtypeStruct((orows, 128), jnp.float32),
        ],
        mesh=mesh,
        compiler_params=pltpu.CompilerParams(
            use_tc_tiling_on_sc=False, needs_layout_passes=False),
        scratch_types=[
            pltpu.VMEM((e_w, F), jnp.float32),
            pltpu.VMEM((e_w, F), jnp.float32),
            pltpu.VMEM((e_w, F), jnp.float32),
            pltpu.VMEM((nch, _CH), jnp.int32),
            pltpu.VMEM((nch, _CH), jnp.int32),
            pltpu.VMEM((e_w,), jnp.float32),
            pltpu.VMEM((e_w,), jnp.float32),
            pltpu.VMEM((e_w,), jnp.float32),
            pltpu.VMEM((_NPAD // 16, F), jnp.float32),
            pltpu.VMEM((_NPAD * F // 16 // 128, 128), jnp.float32),
            pltpu.VMEM_SHARED((_ACCR, F), jnp.float32),
            pltpu.VMEM_SHARED((_ACCR, F), jnp.float32),
            pltpu.VMEM_SHARED((_ACCR, F), jnp.float32),
            pltpu.SemaphoreType.DMA,
            pltpu.SemaphoreType.DMA,
        ],
    )(xTp, idxg, idxs, w0f, w1f, ddf, zx)

    M = np.zeros((4, 4), np.float32)
    M[0, 0] = 1.0
    for t in range(1, 4):
        M[t, t] = 1.0 - _ALPHA
        M[t - 1, t] = _ALPHA
    S32 = jnp.asarray(np.kron(M, np.eye(8, dtype=np.float32)))
    KWS = jnp.kron(jnp.eye(4, dtype=jnp.float32), W[:8, :]) @ S32
    Z32 = jnp.zeros((32, 32), jnp.float32)
    A0 = jnp.concatenate([KWS, Z32], axis=1)
    A1 = jnp.concatenate([Z32, KWS], axis=1)
    w8s = jnp.tile(W[8, :], 4) @ S32
    Ad = jnp.zeros((32, 64), jnp.float32).at[0, :32].set(w8s)
    Ad = Ad.at[1, 32:].set(w8s)
    bt = jnp.broadcast_to(jnp.tile(b, 8)[None, :], (8, 64))

    out = pl.pallas_call(
        functools.partial(_fc_body, nc),
        in_specs=[
            pl.BlockSpec((orows, 128), lambda: (0, 0)),
            pl.BlockSpec((orows, 128), lambda: (0, 0)),
            pl.BlockSpec((orows, 128), lambda: (0, 0)),
            pl.BlockSpec((32, 64), lambda: (0, 0)),
            pl.BlockSpec((32, 64), lambda: (0, 0)),
            pl.BlockSpec((32, 64), lambda: (0, 0)),
            pl.BlockSpec((8, 64), lambda: (0, 0)),
        ],
        out_specs=pl.BlockSpec((T_, _NPAD, 16), lambda: (0, 0, 0)),
        out_shape=jax.ShapeDtypeStruct((T_, _NPAD, 16), jnp.float32),
    )(outx0, outx1, outd, A0, A1, Ad, bt)

    res = out[:, :N, :].reshape(T_, N, 2, 8)
    return res[None]
